# baseline (speedup 1.0000x reference)
import functools

import jax
import jax.numpy as jnp
from jax.experimental import pallas as pl
from jax.experimental.pallas import tpu as pltpu

_L = 32768
_EPS = 1e-10
_PAD_VAL = 1e9


def _body(n, x_ref, q_ref, cx_ref, cy_ref, out_ref, acc_ref):
    i = pl.program_id(0)

    @pl.when(i == 0)
    def _init():
        acc_ref[...] = jnp.zeros_like(acc_ref)

    pos = jax.lax.broadcasted_iota(jnp.int32, (1, _L), 1) + i * _L
    valid = pos < n
    u = jnp.where(valid, x_ref[0:1, :], _PAD_VAL) * q_ref[0:1, 0:1]
    v = jnp.where(valid, x_ref[1:2, :], _PAD_VAL) * q_ref[1:2, 0:1]
    tx = u - cx_ref[...]
    ty = v - cy_ref[...]
    kx = jnp.exp2(tx * (-tx))
    ky = jnp.exp2(ty * (-ty))

    acc_ref[...] += jax.lax.dot_general(
        kx, ky, (((1,), (1,)), ((), ())), preferred_element_type=jnp.float32
    )

    @pl.when(i == pl.num_programs(0) - 1)
    def _fin():
        acc = acc_ref[...]
        out_ref[...] = acc / (jnp.sum(acc) + _EPS)


def kernel(x, bin_edges_x, bin_edges_y):
    n = x.shape[0]
    grid = (n + _L - 1) // _L

    xt = x[:, :2].T
    cx = 0.5 * (bin_edges_x[:-1] + bin_edges_x[1:])
    cy = 0.5 * (bin_edges_y[:-1] + bin_edges_y[1:])
    scale = jnp.sqrt(jnp.log2(jnp.exp(1.0))) / jnp.sqrt(2.0)
    qx = scale / (bin_edges_x[1] - bin_edges_x[0])
    qy = scale / (bin_edges_y[1] - bin_edges_y[0])
    q = jnp.stack([qx, qy]).reshape(2, 1)
    cxs = (cx * qx).reshape(32, 1)
    cys = (cy * qy).reshape(32, 1)

    return pl.pallas_call(
        functools.partial(_body, n),
        grid=(grid,),
        in_specs=[
            pl.BlockSpec((2, _L), lambda i: (0, i)),
            pl.BlockSpec((2, 1), lambda i: (0, 0)),
            pl.BlockSpec((32, 1), lambda i: (0, 0)),
            pl.BlockSpec((32, 1), lambda i: (0, 0)),
        ],
        out_specs=pl.BlockSpec((32, 32), lambda i: (0, 0)),
        out_shape=jax.ShapeDtypeStruct((32, 32), jnp.float32),
        scratch_shapes=[pltpu.VMEM((32, 32), jnp.float32)],
        compiler_params=pltpu.CompilerParams(
            dimension_semantics=("arbitrary",)
        ),
    )(xt, q, cxs, cys)


# speedup vs baseline: 1.1489x; 1.1489x over previous
import jax
import jax.numpy as jnp
from jax.experimental import pallas as pl
from jax.experimental.pallas import tpu as pltpu

_L = 32768
_EPS = 1e-10
_PAD_VAL = 1e9


def _body(xt_ref, q_ref, cx_ref, cy_ref, out_ref, acc_ref):
    i = pl.program_id(0)

    @pl.when(i == 0)
    def _init():
        acc_ref[...] = jnp.zeros_like(acc_ref)

    u = (xt_ref[0:1, :] * q_ref[0:1, 0:1]).astype(jnp.bfloat16)
    v = (xt_ref[1:2, :] * q_ref[1:2, 0:1]).astype(jnp.bfloat16)
    tx = u - cx_ref[...].astype(jnp.bfloat16)
    ty = v - cy_ref[...].astype(jnp.bfloat16)
    kx = jnp.exp2(tx * (-tx))
    ky = jnp.exp2(ty * (-ty))

    acc_ref[...] += jax.lax.dot_general(
        kx, ky, (((1,), (1,)), ((), ())), preferred_element_type=jnp.float32
    )

    @pl.when(i == pl.num_programs(0) - 1)
    def _fin():
        acc = acc_ref[...]
        out_ref[...] = acc / (jnp.sum(acc) + _EPS)


def kernel(x, bin_edges_x, bin_edges_y):
    n = x.shape[0]
    grid = (n + _L - 1) // _L
    npad = grid * _L - n

    xt = jnp.pad(
        x[:, :2].T, ((0, 0), (0, npad)), constant_values=_PAD_VAL
    )
    cx = 0.5 * (bin_edges_x[:-1] + bin_edges_x[1:])
    cy = 0.5 * (bin_edges_y[:-1] + bin_edges_y[1:])
    scale = jnp.sqrt(jnp.log2(jnp.exp(1.0))) / jnp.sqrt(2.0)
    qx = scale / (bin_edges_x[1] - bin_edges_x[0])
    qy = scale / (bin_edges_y[1] - bin_edges_y[0])
    q = jnp.stack([qx, qy]).reshape(2, 1)
    cxs = (cx * qx).reshape(32, 1)
    cys = (cy * qy).reshape(32, 1)

    return pl.pallas_call(
        _body,
        grid=(grid,),
        in_specs=[
            pl.BlockSpec((2, _L), lambda i: (0, i)),
            pl.BlockSpec((2, 1), lambda i: (0, 0)),
            pl.BlockSpec((32, 1), lambda i: (0, 0)),
            pl.BlockSpec((32, 1), lambda i: (0, 0)),
        ],
        out_specs=pl.BlockSpec((32, 32), lambda i: (0, 0)),
        out_shape=jax.ShapeDtypeStruct((32, 32), jnp.float32),
        scratch_shapes=[pltpu.VMEM((32, 32), jnp.float32)],
        compiler_params=pltpu.CompilerParams(
            dimension_semantics=("arbitrary",)
        ),
    )(xt, q, cxs, cys)
